# Initial kernel scaffold; baseline (speedup 1.0000x reference)
import jax
import jax.numpy as jnp
from jax.experimental import pallas as pl


def kernel(x, edge_index, eps, kan0_base, kan0_spline, kan1_base, kan1_spline, bn0_gamma, bn0_beta, bn1_gamma, bn1_beta, clf_w, clf_b):
    raise NotImplementedError("write your pallas kernel here")


# speedup vs baseline: 2.7074x; 2.7074x over previous
import functools

import jax
import jax.numpy as jnp
from jax import lax
from jax.experimental import pallas as pl
from jax.experimental.pallas import tpu as pltpu
from jax.experimental.pallas import tpu_sc as plsc

N_NODES = 10000
N_EDGES = 320000
D = 128
HID = 128
OUT = 10
GRID_SIZE = 5
SPLINE_ORDER = 3
COEF = GRID_SIZE + SPLINE_ORDER

NC = 2
NS = 16
NW = NC * NS
CHUNK = 128
E_PAD = 327680
NCHUNK = E_PAD // (NW * CHUNK)
ACC_ROWS = 10112
ROWS_PER_TILE = ACC_ROWS // NS

KNOTS = [0.4 * i - 2.2 for i in range(GRID_SIZE + 2 * SPLINE_ORDER + 1)]


def _sc_scatter_body(h_hbm, row_hbm, col_hbm, zeros_hbm, out_hbm,
                     row_v, col_v, rows_v, sem, acc_sh):
  c = lax.axis_index("c")
  s = lax.axis_index("s")
  wid = c * NS + s

  pltpu.sync_copy(zeros_hbm.at[pl.ds(s * ROWS_PER_TILE, ROWS_PER_TILE)],
                  acc_sh.at[pl.ds(s * ROWS_PER_TILE, ROWS_PER_TILE)])
  plsc.subcore_barrier()

  pltpu.sync_copy(row_hbm.at[wid], row_v)
  pltpu.sync_copy(col_hbm.at[wid], col_v)

  def body(j, carry):
    pltpu.async_copy(h_hbm.at[col_v.at[j]], rows_v, sem).wait()
    pltpu.sync_copy(rows_v, acc_sh.at[row_v.at[j]], add=True)
    return carry

  lax.fori_loop(0, NCHUNK, body, 0, unroll=False)
  plsc.subcore_barrier()

  pltpu.sync_copy(acc_sh.at[pl.ds(s * ROWS_PER_TILE, ROWS_PER_TILE)],
                  out_hbm.at[c, pl.ds(s * ROWS_PER_TILE, ROWS_PER_TILE)])


@jax.jit
def _sc_scatter(h, row_r, col_r, zeros):
  mesh = plsc.VectorSubcoreMesh(core_axis_name="c", subcore_axis_name="s")
  fn = pl.kernel(
      _sc_scatter_body,
      out_type=jax.ShapeDtypeStruct((NC, ACC_ROWS, D), jnp.float32),
      mesh=mesh,
      scratch_types=[
          pltpu.VMEM((NCHUNK, CHUNK), jnp.int32),
          pltpu.VMEM((NCHUNK, CHUNK), jnp.int32),
          pltpu.VMEM((CHUNK, D), jnp.float32),
          pltpu.SemaphoreType.DMA,
          pltpu.VMEM_SHARED((ACC_ROWS, D), jnp.float32),
      ],
  )
  return fn(h, row_r, col_r, zeros)


def _bspline_bases(x):
  g = KNOTS
  b = [jnp.where((x >= g[j]) & (x < g[j + 1]), 1.0, 0.0).astype(x.dtype)
       for j in range(len(g) - 1)]
  for k in range(1, SPLINE_ORDER + 1):
    nb = []
    for j in range(len(b) - 1):
      left = (x - g[j]) * (1.0 / (g[j + k] - g[j])) * b[j]
      right = (g[j + k + 1] - x) * (1.0 / (g[j + k + 1] - g[j + 1])) * b[j + 1]
      nb.append(left + right)
    b = nb
  return b


def _dense1_body(scale_ref, p0_ref, p1_ref, h_ref, bwt_ref, swt_ref,
                 y_ref, sums_ref):
  pid = pl.program_id(0)
  pooled = p0_ref[...] + p1_ref[...] + scale_ref[0] * h_ref[...]
  silu = pooled * jax.nn.sigmoid(pooled)
  y = jnp.dot(silu, bwt_ref[...], preferred_element_type=jnp.float32)
  bases = _bspline_bases(pooled)
  for j in range(COEF):
    y = y + jnp.dot(bases[j], swt_ref[j], preferred_element_type=jnp.float32)
  y_ref[...] = y

  ssum = jnp.sum(y, axis=0)
  ssq = jnp.sum(y * y, axis=0)
  upd = jnp.concatenate(
      [ssum[None], ssq[None], jnp.zeros((6, y.shape[1]), jnp.float32)], axis=0)

  @pl.when(pid == 0)
  def _():
    sums_ref[...] = jnp.zeros_like(sums_ref)

  sums_ref[...] += upd


def _dense1(scale, p0, p1, h, bwt, swt, block):
  nblk = N_NODES // block
  return pl.pallas_call(
      _dense1_body,
      grid=(nblk,),
      in_specs=[
          pl.BlockSpec(memory_space=pltpu.SMEM),
          pl.BlockSpec((block, D), lambda i: (i, 0)),
          pl.BlockSpec((block, D), lambda i: (i, 0)),
          pl.BlockSpec((block, D), lambda i: (i, 0)),
          pl.BlockSpec((D, HID), lambda i: (0, 0)),
          pl.BlockSpec((COEF, D, HID), lambda i: (0, 0, 0)),
      ],
      out_specs=[
          pl.BlockSpec((block, HID), lambda i: (i, 0)),
          pl.BlockSpec((8, HID), lambda i: (0, 0)),
      ],
      out_shape=[
          jax.ShapeDtypeStruct((N_NODES, HID), jnp.float32),
          jax.ShapeDtypeStruct((8, HID), jnp.float32),
      ],
  )(scale, p0, p1, h, bwt, swt)


def _bn_relu_body(y_ref, sums_ref, gamma_ref, beta_ref, o_ref):
  inv_n = 1.0 / N_NODES
  mean = sums_ref[0:1, :] * inv_n
  var = sums_ref[1:2, :] * inv_n - mean * mean
  inv = lax.rsqrt(var + 1e-5) * gamma_ref[...]
  o_ref[...] = jnp.maximum((y_ref[...] - mean) * inv + beta_ref[...], 0.0)


def _bn_relu(y, sums, gamma, beta, block):
  nblk = N_NODES // block
  return pl.pallas_call(
      _bn_relu_body,
      grid=(nblk,),
      in_specs=[
          pl.BlockSpec((block, HID), lambda i: (i, 0)),
          pl.BlockSpec((8, HID), lambda i: (0, 0)),
          pl.BlockSpec((1, HID), lambda i: (0, 0)),
          pl.BlockSpec((1, HID), lambda i: (0, 0)),
      ],
      out_specs=pl.BlockSpec((block, HID), lambda i: (i, 0)),
      out_shape=jax.ShapeDtypeStruct((N_NODES, HID), jnp.float32),
  )(y, sums, gamma, beta)


def _bn_relu_clf_body(y_ref, sums_ref, gamma_ref, beta_ref, cw_ref, cb_ref,
                      o_ref):
  inv_n = 1.0 / N_NODES
  mean = sums_ref[0:1, :] * inv_n
  var = sums_ref[1:2, :] * inv_n - mean * mean
  inv = lax.rsqrt(var + 1e-5) * gamma_ref[...]
  h = jnp.maximum((y_ref[...] - mean) * inv + beta_ref[...], 0.0)
  o_ref[...] = jnp.dot(h, cw_ref[...], preferred_element_type=jnp.float32) \
      + cb_ref[...]


def _bn_relu_clf(y, sums, gamma, beta, cw_pad, cb_pad, block):
  nblk = N_NODES // block
  return pl.pallas_call(
      _bn_relu_clf_body,
      grid=(nblk,),
      in_specs=[
          pl.BlockSpec((block, HID), lambda i: (i, 0)),
          pl.BlockSpec((8, HID), lambda i: (0, 0)),
          pl.BlockSpec((1, HID), lambda i: (0, 0)),
          pl.BlockSpec((1, HID), lambda i: (0, 0)),
          pl.BlockSpec((HID, 128), lambda i: (0, 0)),
          pl.BlockSpec((1, 128), lambda i: (0, 0)),
      ],
      out_specs=pl.BlockSpec((block, 128), lambda i: (i, 0)),
      out_shape=jax.ShapeDtypeStruct((N_NODES, 128), jnp.float32),
  )(y, sums, gamma, beta, cw_pad, cb_pad)


BLOCK = 1000


def kernel(x, edge_index, eps, kan0_base, kan0_spline, kan1_base, kan1_spline,
           bn0_gamma, bn0_beta, bn1_gamma, bn1_beta, clf_w, clf_b):
  row = edge_index[0].astype(jnp.int32)
  col = edge_index[1].astype(jnp.int32)
  pad = E_PAD - N_EDGES
  row_r = jnp.concatenate(
      [row, jnp.full((pad,), N_NODES, jnp.int32)]).reshape(NW, NCHUNK, CHUNK)
  col_r = jnp.concatenate(
      [col, jnp.zeros((pad,), jnp.int32)]).reshape(NW, NCHUNK, CHUNK)
  zeros = jnp.zeros((ACC_ROWS, D), jnp.float32)

  bwts = [kan0_base.T, kan1_base.T]
  swts = [kan0_spline.transpose(2, 1, 0), kan1_spline.transpose(2, 1, 0)]
  gammas = [bn0_gamma.reshape(1, HID), bn1_gamma.reshape(1, HID)]
  betas = [bn0_beta.reshape(1, HID), bn1_beta.reshape(1, HID)]
  cw_pad = jnp.zeros((HID, 128), jnp.float32).at[:, :OUT].set(clf_w.T)
  cb_pad = jnp.zeros((1, 128), jnp.float32).at[0, :OUT].set(clf_b)

  h = x
  for layer in range(2):
    partials = _sc_scatter(h, row_r, col_r, zeros)[:, :N_NODES]
    scale = (1.0 + eps[layer]).reshape(1)
    y, sums = _dense1(scale, partials[0], partials[1], h,
                      bwts[layer], swts[layer], BLOCK)
    if layer == 0:
      h = _bn_relu(y, sums, gammas[layer], betas[layer], BLOCK)
    else:
      logits_pad = _bn_relu_clf(y, sums, gammas[layer], betas[layer],
                                cw_pad, cb_pad, BLOCK)
  return logits_pad[:, :OUT]
